# baseline (speedup 1.0000x reference)
import functools

import jax
import jax.numpy as jnp
from jax import lax
from jax.experimental import pallas as pl
from jax.experimental.pallas import tpu as pltpu
from jax.experimental.pallas import tpu_sc as plsc

L = 16
NC = 2
NS = 16
NW = NC * NS

_BATCH = 16384
_HIST = 200
_DIM = 32
_RW = _BATCH // NW
_C = 8
_NK = _RW // _C
_CH = _C * _HIST
_GF = _HIST // L


def _sc_body(idx_hbm, sc_hbm, tbl_hbm, out_hbm,
             idx_v0, idx_v1, sc_v0, sc_v1, rows_v0, rows_v1, out_v,
             csem0, csem1, gsem0, gsem1):
  idx_bufs = (idx_v0, idx_v1)
  sc_bufs = (sc_v0, sc_v1)
  rows_bufs = (rows_v0, rows_v1)
  csems = (csem0, csem1)
  gsems = (gsem0, gsem1)

  wid = lax.axis_index("s") * NC + lax.axis_index("c")
  fbase = wid * (_RW * _HIST)

  def copy_start(kk, slot):
    off = fbase + kk * _CH
    pltpu.async_copy(idx_hbm.at[pl.ds(off, _CH)], idx_bufs[slot], csems[slot])
    pltpu.async_copy(sc_hbm.at[pl.ds(off, _CH)], sc_bufs[slot], csems[slot])

  def copy_wait(kk, slot):
    off = fbase + kk * _CH
    pltpu.make_async_copy(idx_hbm.at[pl.ds(off, _CH)], idx_bufs[slot],
                          csems[slot]).wait()
    pltpu.make_async_copy(sc_hbm.at[pl.ds(off, _CH)], sc_bufs[slot],
                          csems[slot]).wait()

  def gather_start(slot):
    pltpu.async_copy(tbl_hbm.at[idx_bufs[slot]], rows_bufs[slot], gsems[slot])

  def gather_wait(slot):
    pltpu.make_async_copy(tbl_hbm.at[idx_bufs[slot]], rows_bufs[slot],
                          gsems[slot]).wait()

  def compute(kk, slot):
    rows = rows_bufs[slot]
    scs = sc_bufs[slot]

    def accum(off, js, accs):
      a0, a1 = accs
      sv = scs[pl.ds(off, L)]
      for j in js:
        s = sv[j]
        v = rows[off + j, :]
        a0 = a0 + lax.bitcast_convert_type(v << 16, jnp.float32) * s
        a1 = a1 + lax.bitcast_convert_type(v & jnp.int32(-65536),
                                           jnp.float32) * s
      return a0, a1

    def row_body(c, _):
      roff = c * _HIST

      def grp_body(g, accs):
        return accum(roff + g * L, range(L), accs)

      z = jnp.zeros((L,), jnp.float32)
      a0, a1 = lax.fori_loop(0, _GF, grp_body, (z, z))
      a0, a1 = accum(roff + _HIST - L, range(L - 8, L), (a0, a1))
      out_v[kk * _C + c, pl.ds(0, L)] = a0
      out_v[kk * _C + c, pl.ds(L, L)] = a1
      return 0

    lax.fori_loop(0, _C, row_body, 0)

  copy_start(0, 0)
  copy_wait(0, 0)
  gather_start(0)
  copy_start(1, 1)

  def outer(i, _):
    for s in range(2):
      kk = i * 2 + s
      slot = s
      nslot = 1 - s
      gather_wait(slot)

      @pl.when(kk + 1 < _NK)
      def _():
        copy_wait(kk + 1, nslot)
        gather_start(nslot)

      compute(kk, slot)

      @pl.when(kk + 2 < _NK)
      def _():
        copy_start(kk + 2, slot)
    return 0

  lax.fori_loop(0, _NK // 2, outer, 0)

  pltpu.sync_copy(out_v, out_hbm.at[pl.ds(wid * _RW, _RW)])


@functools.partial(
    pl.kernel,
    out_type=jax.ShapeDtypeStruct((_BATCH, _DIM), jnp.float32),
    mesh=plsc.VectorSubcoreMesh(core_axis_name="c", subcore_axis_name="s"),
    compiler_params=pltpu.CompilerParams(use_tc_tiling_on_sc=False),
    scratch_types=[
        pltpu.VMEM((_CH,), jnp.int32),
        pltpu.VMEM((_CH,), jnp.int32),
        pltpu.VMEM((_CH,), jnp.float32),
        pltpu.VMEM((_CH,), jnp.float32),
        pltpu.VMEM((_CH, L), jnp.int32),
        pltpu.VMEM((_CH, L), jnp.int32),
        pltpu.VMEM((_RW, _DIM), jnp.float32),
        pltpu.SemaphoreType.DMA,
        pltpu.SemaphoreType.DMA,
        pltpu.SemaphoreType.DMA,
        pltpu.SemaphoreType.DMA,
    ],
)
def _sc_call(idx_hbm, sc_hbm, tbl_hbm, out_hbm, *rest):
  _sc_body(idx_hbm, sc_hbm, tbl_hbm, out_hbm, *rest)


@jax.jit
def kernel(indices, scores, weight):
  wi = lax.bitcast_convert_type(weight, jnp.uint32)
  rne = (wi + jnp.uint32(0x7FFF) + ((wi >> 16) & jnp.uint32(1))) >> 16
  tbl = lax.bitcast_convert_type(rne[:, :L] | (rne[:, L:] << 16), jnp.int32)
  return _sc_call(indices.reshape(-1), scores.reshape(-1), tbl)


# speedup vs baseline: 2.1295x; 2.1295x over previous
import functools

import jax
import jax.numpy as jnp
from jax import lax
from jax.experimental import pallas as pl
from jax.experimental.pallas import tpu as pltpu
from jax.experimental.pallas import tpu_sc as plsc

L = 16
NC = 2
NS = 16
NW = NC * NS

_BATCH = 16384
_HIST = 200
_DIM = 32
_RW = _BATCH // NW
_C = 8
_NK = _RW // _C
_CH = _C * _HIST
_GF = _HIST // L


def _sc_body(idx_hbm, sc_hbm, tbl_hbm, out_hbm,
             idx_v0, idx_v1, sc_v0, sc_v1, rows_v0, rows_v1, out_v,
             csem0, csem1, gsem0, gsem1):
  idx_bufs = (idx_v0, idx_v1)
  sc_bufs = (sc_v0, sc_v1)
  rows_bufs = (rows_v0, rows_v1)
  csems = (csem0, csem1)
  gsems = (gsem0, gsem1)

  wid = lax.axis_index("s") * NC + lax.axis_index("c")
  fbase = wid * (_RW * _HIST)

  def copy_start(kk, slot):
    off = fbase + kk * _CH
    pltpu.async_copy(idx_hbm.at[pl.ds(off, _CH)], idx_bufs[slot], csems[slot])
    pltpu.async_copy(sc_hbm.at[pl.ds(off, _CH)], sc_bufs[slot], csems[slot])

  def copy_wait(kk, slot):
    off = fbase + kk * _CH
    pltpu.make_async_copy(idx_hbm.at[pl.ds(off, _CH)], idx_bufs[slot],
                          csems[slot]).wait()
    pltpu.make_async_copy(sc_hbm.at[pl.ds(off, _CH)], sc_bufs[slot],
                          csems[slot]).wait()

  def gather_start(slot):
    pltpu.async_copy(tbl_hbm.at[idx_bufs[slot]], rows_bufs[slot], gsems[slot])

  def gather_wait(slot):
    pltpu.make_async_copy(tbl_hbm.at[idx_bufs[slot]], rows_bufs[slot],
                          gsems[slot]).wait()

  def compute(kk, slot):
    rows = rows_bufs[slot]
    scs = sc_bufs[slot]

    def accum(off, js, accs):
      a0, a1 = accs
      sv = scs[pl.ds(off, L)]
      for j in js:
        s = sv[j]
        a0 = a0 + rows[off + j, pl.ds(0, L)] * s
        a1 = a1 + rows[off + j, pl.ds(L, L)] * s
      return a0, a1

    def row_body(c, _):
      roff = c * _HIST

      def grp_body(g, accs):
        return accum(roff + g * L, range(L), accs)

      z = jnp.zeros((L,), jnp.float32)
      a0, a1 = lax.fori_loop(0, _GF, grp_body, (z, z))
      a0, a1 = accum(roff + _HIST - L, range(L - 8, L), (a0, a1))
      out_v[kk * _C + c, pl.ds(0, L)] = a0
      out_v[kk * _C + c, pl.ds(L, L)] = a1
      return 0

    lax.fori_loop(0, _C, row_body, 0)

  copy_start(0, 0)
  copy_wait(0, 0)
  gather_start(0)
  copy_start(1, 1)

  def outer(i, _):
    for s in range(2):
      kk = i * 2 + s
      slot = s
      nslot = 1 - s
      gather_wait(slot)

      @pl.when(kk + 1 < _NK)
      def _():
        copy_wait(kk + 1, nslot)
        gather_start(nslot)

      compute(kk, slot)

      @pl.when(kk + 2 < _NK)
      def _():
        copy_start(kk + 2, slot)
    return 0

  lax.fori_loop(0, _NK // 2, outer, 0)

  pltpu.sync_copy(out_v, out_hbm.at[pl.ds(wid * _RW, _RW)])


@functools.partial(
    pl.kernel,
    out_type=jax.ShapeDtypeStruct((_BATCH, _DIM), jnp.float32),
    mesh=plsc.VectorSubcoreMesh(core_axis_name="c", subcore_axis_name="s"),
    compiler_params=pltpu.CompilerParams(use_tc_tiling_on_sc=False),
    scratch_types=[
        pltpu.VMEM((_CH,), jnp.int32),
        pltpu.VMEM((_CH,), jnp.int32),
        pltpu.VMEM((_CH,), jnp.float32),
        pltpu.VMEM((_CH,), jnp.float32),
        pltpu.VMEM((_CH, _DIM), jnp.float32),
        pltpu.VMEM((_CH, _DIM), jnp.float32),
        pltpu.VMEM((_RW, _DIM), jnp.float32),
        pltpu.SemaphoreType.DMA,
        pltpu.SemaphoreType.DMA,
        pltpu.SemaphoreType.DMA,
        pltpu.SemaphoreType.DMA,
    ],
)
def _sc_call(idx_hbm, sc_hbm, tbl_hbm, out_hbm, *rest):
  _sc_body(idx_hbm, sc_hbm, tbl_hbm, out_hbm, *rest)


@jax.jit
def kernel(indices, scores, weight):
  return _sc_call(indices.reshape(-1), scores.reshape(-1), weight)
